# baseline (speedup 1.0000x reference)
import functools

import jax
import jax.numpy as jnp
from jax import lax
from jax.experimental import pallas as pl
from jax.experimental.pallas import tpu as pltpu
from jax.experimental.pallas import tpu_sc as plsc

VOCAB = 100000
HID = 1024
BATCH = 4
SEQ = 4096

NC, NS = 2, 16
NW = NC * NS
B = BATCH * SEQ
B_PER_W = B // NW
NCHAIN = 2
ROWS_PER_CHAIN = B_PER_W // NCHAIN
C = 24
NBUF = 2
_CHUNKS = [C] * (ROWS_PER_CHAIN // C) + (
    [ROWS_PER_CHAIN % C] if ROWS_PER_CHAIN % C else [])
_OFFS = [sum(_CHUNKS[:i]) for i in range(len(_CHUNKS))]
NCHUNK = len(_CHUNKS)


def _make_kernel():
    mesh = plsc.VectorSubcoreMesh(core_axis_name="c", subcore_axis_name="s")

    @functools.partial(
        pl.kernel,
        mesh=mesh,
        out_type=jax.ShapeDtypeStruct((B, HID), jnp.float32),
        scratch_types=[
            pltpu.VMEM((B_PER_W,), jnp.int32),
            pltpu.VMEM((NCHAIN, NBUF, C, HID), jnp.float32),
        ] + [pltpu.SemaphoreType.DMA] * (2 * NCHAIN * NBUF),
    )
    def k(table_hbm, idx_hbm, out_hbm, idx_v, rows_v, *sems):
        gsem = sems[:NCHAIN * NBUF]
        ssem = sems[NCHAIN * NBUF:]
        wid = lax.axis_index("s") * NC + lax.axis_index("c")
        base = wid * B_PER_W
        pltpu.sync_copy(idx_hbm.at[wid], idx_v)

        def gather(ch, i, b):
            c = _CHUNKS[i]
            off = ch * ROWS_PER_CHAIN + _OFFS[i]
            return pltpu.async_copy(
                table_hbm.at[idx_v.at[pl.ds(off, c)]],
                rows_v.at[ch].at[b].at[pl.ds(0, c)], gsem[ch * NBUF + b])

        g = [{} for _ in range(NCHAIN)]
        s = [{} for _ in range(NCHAIN)]
        for i in range(NBUF):
            for ch in range(NCHAIN):
                g[ch][i] = gather(ch, i, i)
        for i in range(NCHUNK):
            b = i % NBUF
            for ch in range(NCHAIN):
                c = _CHUNKS[i]
                off = ch * ROWS_PER_CHAIN + _OFFS[i]
                g[ch][i].wait()
                s[ch][i] = pltpu.async_copy(
                    rows_v.at[ch].at[b].at[pl.ds(0, c)],
                    out_hbm.at[pl.ds(base + off, c)], ssem[ch * NBUF + b])
            for ch in range(NCHAIN):
                j = i + NBUF
                if j < NCHUNK:
                    s[ch][i].wait()
                    g[ch][j] = gather(ch, j, b)
        for i in range(max(0, NCHUNK - NBUF), NCHUNK):
            for ch in range(NCHAIN):
                s[ch][i].wait()

    return k


_sc_gather = _make_kernel()


def kernel(token_ids, W):
    idx2 = token_ids.reshape(NW, B_PER_W)
    out = _sc_gather(W, idx2)
    return out.reshape(BATCH, SEQ, HID)


# speedup vs baseline: 1.0083x; 1.0083x over previous
import functools

import jax
import jax.numpy as jnp
from jax import lax
from jax.experimental import pallas as pl
from jax.experimental.pallas import tpu as pltpu
from jax.experimental.pallas import tpu_sc as plsc

VOCAB = 100000
HID = 1024
BATCH = 4
SEQ = 4096

NC, NS = 2, 16
NW = NC * NS
B = BATCH * SEQ
B_PER_W = B // NW
C = 32
NBUF = 3
NCHUNK = B_PER_W // C


def _make_kernel():
    mesh = plsc.VectorSubcoreMesh(core_axis_name="c", subcore_axis_name="s")

    @functools.partial(
        pl.kernel,
        mesh=mesh,
        out_type=jax.ShapeDtypeStruct((B, HID), jnp.float32),
        scratch_types=[
            pltpu.VMEM((NCHUNK, C), jnp.int32),
            pltpu.VMEM((NBUF, C, HID), jnp.float32),
        ] + [pltpu.SemaphoreType.DMA] * (2 * NBUF),
    )
    def k(table_hbm, idx_hbm, out_hbm, idx_v, rows_v, *sems):
        gsem = sems[:NBUF]
        ssem = sems[NBUF:]
        wid = lax.axis_index("s") * NC + lax.axis_index("c")
        base = wid * B_PER_W
        pltpu.sync_copy(idx_hbm.at[wid], idx_v)

        def gather(i, b):
            return pltpu.async_copy(
                table_hbm.at[idx_v.at[i]], rows_v.at[b], gsem[b])

        g = {}
        s = {}
        for i in range(NBUF):
            g[i] = gather(i, i)
        for i in range(NCHUNK):
            b = i % NBUF
            g[i].wait()
            s[i] = pltpu.async_copy(
                rows_v.at[b], out_hbm.at[pl.ds(base + i * C, C)], ssem[b])
            j = i + NBUF
            if j < NCHUNK:
                s[i].wait()
                g[j] = gather(j, b)
        for i in range(max(0, NCHUNK - NBUF), NCHUNK):
            s[i].wait()

    return k


_sc_gather = _make_kernel()


def kernel(token_ids, W):
    idx3 = token_ids.reshape(NW, NCHUNK, C)
    out = _sc_gather(W, idx3)
    return out.reshape(BATCH, SEQ, HID)
